# baseline (speedup 1.0000x reference)
import functools

import jax
import jax.numpy as jnp
from jax import lax
from jax.experimental import pallas as pl
from jax.experimental.pallas import tpu as pltpu
from jax.experimental.pallas import tpu_sc as plsc

K = 8
B = 16
S = 8192
C = 128
NC = 2
LANES = 16
NG = 4
CH_HALF = NG * LANES
CHUNK = 512
NCHUNK = S // CHUNK

WIN = 8
NWIN = CHUNK // WIN

_NET8 = (
    (0, 1), (2, 3), (4, 5), (6, 7),
    (0, 2), (1, 3), (4, 6), (5, 7),
    (1, 2), (5, 6), (0, 4), (3, 7),
    (1, 5), (2, 6),
    (1, 4), (3, 6),
    (2, 4), (3, 5),
    (3, 4),
)
_BITONIC8 = (
    (0, 4), (1, 5), (2, 6), (3, 7),
    (0, 2), (1, 3), (4, 6), (5, 7),
    (0, 1), (2, 3), (4, 5), (6, 7),
)


def _ce(b, i, j):
    hi = jnp.maximum(b[i], b[j])
    lo = jnp.minimum(b[i], b[j])
    b[i] = hi
    b[j] = lo


_mesh = plsc.VectorSubcoreMesh(core_axis_name="c", subcore_axis_name="s")


@functools.partial(
    pl.kernel,
    out_type=jax.ShapeDtypeStruct((B, C * K), jnp.float32),
    mesh=_mesh,
    scratch_types=[
        pltpu.VMEM((CHUNK, CH_HALF), jnp.float32),
        pltpu.VMEM((CHUNK, CH_HALF), jnp.float32),
        pltpu.VMEM((CH_HALF * K,), jnp.float32),
        pltpu.VMEM((NG * K * LANES,), jnp.float32),
        pltpu.SemaphoreType.DMA,
        pltpu.SemaphoreType.DMA,
    ],
    compiler_params=pltpu.CompilerParams(
        use_tc_tiling_on_sc=False, needs_layout_passes=False
    ),
)
def _topk_sc(x_hbm, out_hbm, buf0, buf1, obuf, st_ref, sem0, sem1):
    wid = lax.axis_index("s") * NC + lax.axis_index("c")
    b = wid // 2
    ch0 = (wid % 2) * CH_HALF

    neg = jnp.full((LANES,), -jnp.inf, dtype=jnp.float32)
    for g in range(NG):
        for j in range(K):
            st_ref[pl.ds((g * K + j) * LANES, LANES)] = neg

    bufs = (buf0, buf1)
    sems = (sem0, sem1)
    copies = [None, None]

    def start(i):
        copies[i % 2] = pltpu.async_copy(
            x_hbm.at[b, pl.ds(i * CHUNK, CHUNK), pl.ds(ch0, CH_HALF)],
            bufs[i % 2],
            sems[i % 2],
        )

    start(0)
    for chunk in range(NCHUNK):
        copies[chunk % 2].wait()
        if chunk + 1 < NCHUNK:
            start(chunk + 1)
        buf = bufs[chunk % 2]

        def body(w, c, buf=buf):
            for g in range(NG):
                wb = [
                    buf[w * WIN + r, pl.ds(g * LANES, LANES)]
                    for r in range(WIN)
                ]
                m0 = jnp.maximum(wb[0], wb[1])
                m1 = jnp.maximum(wb[2], wb[3])
                m2 = jnp.maximum(wb[4], wb[5])
                m3 = jnp.maximum(wb[6], wb[7])
                wmax = jnp.maximum(jnp.maximum(m0, m1), jnp.maximum(m2, m3))
                thr = st_ref[pl.ds((g * K + K - 1) * LANES, LANES)]
                cnt = plsc.all_reduce_population_count(wmax > thr)

                @pl.when(cnt[0] > 0)
                def _(g=g, wb=wb):
                    ts = [
                        st_ref[pl.ds((g * K + i) * LANES, LANES)]
                        for i in range(K)
                    ]
                    wb2 = list(wb)
                    for (i, j) in _NET8:
                        _ce(wb2, i, j)
                    ts = [jnp.maximum(ts[i], wb2[K - 1 - i]) for i in range(K)]
                    for (i, j) in _BITONIC8:
                        _ce(ts, i, j)
                    for i in range(K):
                        st_ref[pl.ds((g * K + i) * LANES, LANES)] = ts[i]

            return c

        lax.fori_loop(0, NWIN, body, 0)

    lane = lax.iota(jnp.int32, LANES)
    for g in range(NG):
        for j in range(K):
            idx = lane * K + (g * LANES * K + j)
            plsc.store_scatter(obuf, [idx], st_ref[pl.ds((g * K + j) * LANES, LANES)])
    pltpu.sync_copy(obuf, out_hbm.at[b, pl.ds(ch0 * K, CH_HALF * K)])


def kernel(inputs):
    x = inputs.reshape(B, S, C)
    return _topk_sc(x)


# speedup vs baseline: 1.2306x; 1.2306x over previous
import functools

import jax
import jax.numpy as jnp
from jax import lax
from jax.experimental import pallas as pl
from jax.experimental.pallas import tpu as pltpu
from jax.experimental.pallas import tpu_sc as plsc

K = 8
B = 16
S = 8192
C = 128
NC = 2
LANES = 16
NG = 4
CH_HALF = NG * LANES
CHUNK = 512
NCHUNK = S // CHUNK

WIN = 8
NWIN = CHUNK // WIN

_NET8 = (
    (0, 1), (2, 3), (4, 5), (6, 7),
    (0, 2), (1, 3), (4, 6), (5, 7),
    (1, 2), (5, 6), (0, 4), (3, 7),
    (1, 5), (2, 6),
    (1, 4), (3, 6),
    (2, 4), (3, 5),
    (3, 4),
)
_BITONIC8 = (
    (0, 4), (1, 5), (2, 6), (3, 7),
    (0, 2), (1, 3), (4, 6), (5, 7),
    (0, 1), (2, 3), (4, 5), (6, 7),
)


def _ce(b, i, j):
    hi = jnp.maximum(b[i], b[j])
    lo = jnp.minimum(b[i], b[j])
    b[i] = hi
    b[j] = lo


_mesh = plsc.VectorSubcoreMesh(core_axis_name="c", subcore_axis_name="s")


@functools.partial(
    pl.kernel,
    out_type=jax.ShapeDtypeStruct((B, C * K), jnp.float32),
    mesh=_mesh,
    scratch_types=[
        pltpu.VMEM((CHUNK, CH_HALF), jnp.float32),
        pltpu.VMEM((CHUNK, CH_HALF), jnp.float32),
        pltpu.VMEM((CH_HALF * K,), jnp.float32),
        pltpu.VMEM((NG * K * LANES,), jnp.float32),
        pltpu.VMEM((NG * NWIN + LANES,), jnp.int32),
        pltpu.SemaphoreType.DMA,
        pltpu.SemaphoreType.DMA,
    ],
    compiler_params=pltpu.CompilerParams(
        use_tc_tiling_on_sc=False, needs_layout_passes=False
    ),
)
def _topk_sc(x_hbm, out_hbm, buf0, buf1, obuf, st_ref, wl_ref, sem0, sem1):
    wid = lax.axis_index("s") * NC + lax.axis_index("c")
    b = wid // 2
    ch0 = (wid % 2) * CH_HALF

    neg = jnp.full((LANES,), -jnp.inf, dtype=jnp.float32)
    for g in range(NG):
        for j in range(K):
            st_ref[pl.ds((g * K + j) * LANES, LANES)] = neg

    lane = lax.iota(jnp.int32, LANES)
    lane0 = lane == 0
    one = jnp.full((LANES,), 1, dtype=jnp.int32)

    def process_chunk(buf):
        for g in range(NG):
            thr = st_ref[pl.ds((g * K + K - 1) * LANES, LANES)]

            def p1(w, cur, g=g, thr=thr, buf=buf):
                wb = [
                    buf[w * WIN + r, pl.ds(g * LANES, LANES)]
                    for r in range(WIN)
                ]
                m0 = jnp.maximum(wb[0], wb[1])
                m1 = jnp.maximum(wb[2], wb[3])
                m2 = jnp.maximum(wb[4], wb[5])
                m3 = jnp.maximum(wb[6], wb[7])
                wmax = jnp.maximum(jnp.maximum(m0, m1), jnp.maximum(m2, m3))
                cnt = plsc.all_reduce_population_count(wmax > thr)
                trig = cnt > 0
                wvec = jnp.full((LANES,), w, dtype=jnp.int32)
                plsc.store_scatter(wl_ref, [cur], wvec, mask=trig & lane0)
                return cur + jnp.where(trig, one, 0)

            cur0 = jnp.full((LANES,), g * NWIN, dtype=jnp.int32)
            cur = lax.fori_loop(0, NWIN, p1, cur0)
            n_g = cur[0]

            ts = tuple(
                st_ref[pl.ds((g * K + i) * LANES, LANES)] for i in range(K)
            )

            def p2(ii, ts, buf=buf):
                w = wl_ref[pl.ds(ii, LANES)][0]
                wb = [
                    buf[w * WIN + r, pl.ds(g * LANES, LANES)]
                    for r in range(WIN)
                ]
                for (i, j) in _NET8:
                    _ce(wb, i, j)
                ts = [jnp.maximum(ts[i], wb[K - 1 - i]) for i in range(K)]
                for (i, j) in _BITONIC8:
                    _ce(ts, i, j)
                return tuple(ts)

            ts = lax.fori_loop(jnp.int32(g * NWIN), n_g, p2, ts)
            for i in range(K):
                st_ref[pl.ds((g * K + i) * LANES, LANES)] = ts[i]

    def copy_in(c, buf, sem):
        return pltpu.async_copy(
            x_hbm.at[b, pl.ds(c * CHUNK, CHUNK), pl.ds(ch0, CH_HALF)],
            buf,
            sem,
        )

    copy_in(0, buf0, sem0)
    copy_in(1, buf1, sem1)

    def chunk_pair(i, c):
        for par, (buf, sem) in enumerate(((buf0, sem0), (buf1, sem1))):
            cchunk = i * 2 + par
            pltpu.make_async_copy(
                x_hbm.at[b, pl.ds(0, CHUNK), pl.ds(ch0, CH_HALF)], buf, sem
            ).wait()
            process_chunk(buf)

            @pl.when(cchunk + 2 < NCHUNK)
            def _(cchunk=cchunk, buf=buf, sem=sem):
                copy_in(cchunk + 2, buf, sem)

        return c

    lax.fori_loop(0, NCHUNK // 2, chunk_pair, 0)

    lane = lax.iota(jnp.int32, LANES)
    for g in range(NG):
        for j in range(K):
            idx = lane * K + (g * LANES * K + j)
            plsc.store_scatter(obuf, [idx], st_ref[pl.ds((g * K + j) * LANES, LANES)])
    pltpu.sync_copy(obuf, out_hbm.at[b, pl.ds(ch0 * K, CH_HALF * K)])


def kernel(inputs):
    x = inputs.reshape(B, S, C)
    return _topk_sc(x)


# speedup vs baseline: 1.6270x; 1.3220x over previous
import functools

import jax
import jax.numpy as jnp
from jax import lax
from jax.experimental import pallas as pl
from jax.experimental.pallas import tpu as pltpu
from jax.experimental.pallas import tpu_sc as plsc

K = 8
B = 16
S = 8192
C = 128
NC = 2
LANES = 16
N_TEC = 32
CHUNK = 512
NCHUNK = S // CHUNK
WIN = 8
NWIN = CHUNK // WIN

B_TC = 8
B_SC = B - B_TC

_NET8 = (
    (0, 1), (2, 3), (4, 5), (6, 7),
    (0, 2), (1, 3), (4, 6), (5, 7),
    (1, 2), (5, 6), (0, 4), (3, 7),
    (1, 5), (2, 6),
    (1, 4), (3, 6),
    (2, 4), (3, 5),
    (3, 4),
)
_BITONIC8 = (
    (0, 4), (1, 5), (2, 6), (3, 7),
    (0, 2), (1, 3), (4, 6), (5, 7),
    (0, 1), (2, 3), (4, 5), (6, 7),
)


def _ce(b, i, j):
    hi = jnp.maximum(b[i], b[j])
    lo = jnp.minimum(b[i], b[j])
    b[i] = hi
    b[j] = lo


def _merge_window(wb, st):
    for (i, j) in _NET8:
        _ce(wb, i, j)
    ts = [jnp.maximum(st[i], wb[K - 1 - i]) for i in range(K)]
    for (i, j) in _BITONIC8:
        _ce(ts, i, j)
    return tuple(ts)


_mesh = plsc.VectorSubcoreMesh(core_axis_name="c", subcore_axis_name="s")


def _make_sc(bs):
    ch_div = N_TEC // bs
    ch_w = C // ch_div
    ng = ch_w // LANES

    @functools.partial(
        pl.kernel,
        out_type=jax.ShapeDtypeStruct((bs, C * K), jnp.float32),
        mesh=_mesh,
        scratch_types=[
            pltpu.VMEM((CHUNK, ch_w), jnp.float32),
            pltpu.VMEM((CHUNK, ch_w), jnp.float32),
            pltpu.VMEM((ch_w * K,), jnp.float32),
            pltpu.SemaphoreType.DMA,
            pltpu.SemaphoreType.DMA,
        ],
        compiler_params=pltpu.CompilerParams(
            use_tc_tiling_on_sc=False, needs_layout_passes=False
        ),
    )
    def _topk_sc(x_hbm, out_hbm, buf0, buf1, obuf, sem0, sem1):
        wid = lax.axis_index("s") * NC + lax.axis_index("c")
        b = wid // ch_div
        ch0 = (wid % ch_div) * ch_w

        neg = jnp.full((LANES,), -jnp.inf, dtype=jnp.float32)
        states = tuple(tuple(neg for _ in range(K)) for _ in range(ng))

        bufs = (buf0, buf1)
        sems = (sem0, sem1)
        copies = [None, None]

        def start(i):
            copies[i % 2] = pltpu.async_copy(
                x_hbm.at[b, pl.ds(i * CHUNK, CHUNK), pl.ds(ch0, ch_w)],
                bufs[i % 2],
                sems[i % 2],
            )

        start(0)
        for chunk in range(NCHUNK):
            copies[chunk % 2].wait()
            if chunk + 1 < NCHUNK:
                start(chunk + 1)
            buf = bufs[chunk % 2]

            new_states = []
            for half in range(0, ng, 2):
                def body(w, st, buf=buf, half=half):
                    out_st = []
                    for gg in range(2):
                        g = half + gg
                        wb = [
                            buf[w * WIN + r, pl.ds(g * LANES, LANES)]
                            for r in range(WIN)
                        ]
                        out_st.append(_merge_window(wb, st[gg]))
                    return tuple(out_st)

                pair = (states[half], states[half + 1])
                pair = lax.fori_loop(0, NWIN, body, pair)
                new_states.extend(pair)
            states = tuple(new_states)

        lane = lax.iota(jnp.int32, LANES)
        for g in range(ng):
            for j in range(K):
                idx = lane * K + (g * LANES * K + j)
                plsc.store_scatter(obuf, [idx], states[g][j])
        pltpu.sync_copy(obuf, out_hbm.at[b, pl.ds(ch0 * K, ch_w * K)])

    return _topk_sc


_topk_sc_part = _make_sc(B_SC)


def _tc_body(x_ref, o_ref):
    def win(w, st):
        base = pl.multiple_of(w * 64, 64)
        wb = [x_ref[0, pl.ds(base + r * 8, 8), :] for r in range(WIN)]
        return _merge_window(wb, st)

    ninf = jnp.full((8, C), -jnp.inf, dtype=jnp.float32)
    st = lax.fori_loop(0, S // 64, win, tuple(ninf for _ in range(K)))

    cand = jnp.concatenate(st, axis=0)
    rowi = lax.broadcasted_iota(jnp.int32, (8 * K, C), 0)
    outs = []
    for _ in range(K):
        m = jnp.max(cand, axis=0, keepdims=True)
        eq = cand == m
        fi = jnp.min(jnp.where(eq, rowi, 8 * K), axis=0, keepdims=True)
        cand = jnp.where(rowi == fi, -jnp.inf, cand)
        outs.append(m)
    o_ref[0, :, :] = jnp.concatenate(outs, axis=0)


def _topk_tc(x):
    bs = x.shape[0]
    out = pl.pallas_call(
        _tc_body,
        grid=(bs,),
        in_specs=[pl.BlockSpec((1, S, C), lambda i: (i, 0, 0))],
        out_specs=pl.BlockSpec((1, K, C), lambda i: (i, 0, 0)),
        out_shape=jax.ShapeDtypeStruct((bs, K, C), jnp.float32),
    )(x)
    return out.transpose(0, 2, 1).reshape(bs, C * K)


def kernel(inputs):
    x = inputs.reshape(B, S, C)
    out_tc = _topk_tc(x[:B_TC])
    out_sc = _topk_sc_part(x[B_TC:])
    return jnp.concatenate([out_tc, out_sc], axis=0)


# speedup vs baseline: 1.7379x; 1.0682x over previous
import functools

import jax
import jax.numpy as jnp
from jax import lax
from jax.experimental import pallas as pl
from jax.experimental.pallas import tpu as pltpu
from jax.experimental.pallas import tpu_sc as plsc

K = 8
B = 16
S = 8192
C = 128
NC = 2
LANES = 16
N_TEC = 32
CHUNK = 512
NCHUNK = S // CHUNK
WIN = 8
NWIN = CHUNK // WIN

B_TC = 0
B_SC = B - B_TC

_NET8 = (
    (0, 1), (2, 3), (4, 5), (6, 7),
    (0, 2), (1, 3), (4, 6), (5, 7),
    (1, 2), (5, 6), (0, 4), (3, 7),
    (1, 5), (2, 6),
    (1, 4), (3, 6),
    (2, 4), (3, 5),
    (3, 4),
)
_BITONIC8 = (
    (0, 4), (1, 5), (2, 6), (3, 7),
    (0, 2), (1, 3), (4, 6), (5, 7),
    (0, 1), (2, 3), (4, 5), (6, 7),
)


def _ce(b, i, j):
    hi = jnp.maximum(b[i], b[j])
    lo = jnp.minimum(b[i], b[j])
    b[i] = hi
    b[j] = lo


def _merge_window(wb, st):
    for (i, j) in _NET8:
        _ce(wb, i, j)
    ts = [jnp.maximum(st[i], wb[K - 1 - i]) for i in range(K)]
    for (i, j) in _BITONIC8:
        _ce(ts, i, j)
    return tuple(ts)


_mesh = plsc.VectorSubcoreMesh(core_axis_name="c", subcore_axis_name="s")


def _make_sc(bs):
    ch_div = N_TEC // bs
    ch_w = C // ch_div
    ng = ch_w // LANES

    @functools.partial(
        pl.kernel,
        out_type=jax.ShapeDtypeStruct((bs, C * K), jnp.float32),
        mesh=_mesh,
        scratch_types=[
            pltpu.VMEM((CHUNK, ch_w), jnp.float32),
            pltpu.VMEM((CHUNK, ch_w), jnp.float32),
            pltpu.VMEM((ch_w * K,), jnp.float32),
            pltpu.SemaphoreType.DMA,
            pltpu.SemaphoreType.DMA,
        ],
        compiler_params=pltpu.CompilerParams(
            use_tc_tiling_on_sc=False, needs_layout_passes=False
        ),
    )
    def _topk_sc(x_hbm, out_hbm, buf0, buf1, obuf, sem0, sem1):
        wid = lax.axis_index("s") * NC + lax.axis_index("c")
        b = wid // ch_div
        ch0 = (wid % ch_div) * ch_w

        neg = jnp.full((LANES,), -jnp.inf, dtype=jnp.float32)
        states = tuple(tuple(neg for _ in range(K)) for _ in range(ng))

        bufs = (buf0, buf1)
        sems = (sem0, sem1)
        copies = [None, None]

        def start(i):
            copies[i % 2] = pltpu.async_copy(
                x_hbm.at[b, pl.ds(i * CHUNK, CHUNK), pl.ds(ch0, ch_w)],
                bufs[i % 2],
                sems[i % 2],
            )

        start(0)
        for chunk in range(NCHUNK):
            copies[chunk % 2].wait()
            if chunk + 1 < NCHUNK:
                start(chunk + 1)
            buf = bufs[chunk % 2]

            new_states = []
            for half in range(0, ng, 2):
                def body(w, st, buf=buf, half=half):
                    out_st = []
                    for gg in range(2):
                        g = half + gg
                        wb = [
                            buf[w * WIN + r, pl.ds(g * LANES, LANES)]
                            for r in range(WIN)
                        ]
                        out_st.append(_merge_window(wb, st[gg]))
                    return tuple(out_st)

                pair = (states[half], states[half + 1])
                pair = lax.fori_loop(0, NWIN, body, pair)
                new_states.extend(pair)
            states = tuple(new_states)

        lane = lax.iota(jnp.int32, LANES)
        for g in range(ng):
            for j in range(K):
                idx = lane * K + (g * LANES * K + j)
                plsc.store_scatter(obuf, [idx], states[g][j])
        pltpu.sync_copy(obuf, out_hbm.at[b, pl.ds(ch0 * K, ch_w * K)])

    return _topk_sc


_topk_sc_part = _make_sc(B_SC)


def _tc_body(x_ref, o_ref):
    def win(w, st):
        base = pl.multiple_of(w * 64, 64)
        wb = [x_ref[0, pl.ds(base + r * 8, 8), :] for r in range(WIN)]
        return _merge_window(wb, st)

    ninf = jnp.full((8, C), -jnp.inf, dtype=jnp.float32)
    st = lax.fori_loop(0, S // 64, win, tuple(ninf for _ in range(K)))

    cand = jnp.concatenate(st, axis=0)
    rowi = lax.broadcasted_iota(jnp.int32, (8 * K, C), 0)
    outs = []
    for _ in range(K):
        m = jnp.max(cand, axis=0, keepdims=True)
        eq = cand == m
        fi = jnp.min(jnp.where(eq, rowi, 8 * K), axis=0, keepdims=True)
        cand = jnp.where(rowi == fi, -jnp.inf, cand)
        outs.append(m)
    o_ref[0, :, :] = jnp.concatenate(outs, axis=0)


def _topk_tc(x):
    bs = x.shape[0]
    out = pl.pallas_call(
        _tc_body,
        grid=(bs,),
        in_specs=[pl.BlockSpec((1, S, C), lambda i: (i, 0, 0))],
        out_specs=pl.BlockSpec((1, K, C), lambda i: (i, 0, 0)),
        out_shape=jax.ShapeDtypeStruct((bs, K, C), jnp.float32),
    )(x)
    return out.transpose(0, 2, 1).reshape(bs, C * K)


def kernel(inputs):
    x = inputs.reshape(B, S, C)
    if B_TC == 0:
        return _topk_sc_part(x)
    out_tc = _topk_tc(x[:B_TC])
    out_sc = _topk_sc_part(x[B_TC:])
    return jnp.concatenate([out_tc, out_sc], axis=0)
